# baseline (speedup 1.0000x reference)
import functools

import jax
import jax.numpy as jnp
from jax import lax
from jax.experimental import pallas as pl
from jax.experimental.pallas import tpu as pltpu
from jax.experimental.pallas import tpu_sc as plsc

BLK = 128


@functools.lru_cache(maxsize=None)
def _make_gather(s: int, b: int, d: int):
    info = plsc.get_sparse_core_info()
    nc = info.num_cores
    nw = nc * info.num_subcores
    assert b == nw * BLK and d % 16 == 0
    mesh = plsc.VectorSubcoreMesh(core_axis_name="c", subcore_axis_name="s")

    @functools.partial(
        pl.kernel,
        mesh=mesh,
        out_type=jax.ShapeDtypeStruct((s, d, b), jnp.float32),
        compiler_params=pltpu.CompilerParams(
            use_tc_tiling_on_sc=False, needs_layout_passes=False
        ),
        scratch_types=[
            pltpu.VMEM((s, BLK), jnp.int32),
            pltpu.VMEM((2, BLK, d), jnp.float32),
            pltpu.VMEM((2, d, BLK), jnp.float32),
            pltpu.SemaphoreType.DMA((2,)),
            pltpu.SemaphoreType.DMA((2,)),
        ],
    )
    def gather_kernel(idx_hbm, table_hbm, out_hbm, idx_v, rows_v, rowst_v, gsem, osem):
        wid = lax.axis_index("s") * nc + lax.axis_index("c")
        col0 = wid * BLK
        pltpu.sync_copy(idx_hbm.at[:, pl.ds(col0, BLK)], idx_v)

        iota = lax.iota(jnp.int32, 16)

        def fire_gather(l, a):
            pltpu.async_copy(
                table_hbm.at[idx_v.at[l]], rows_v.at[a], gsem.at[a]
            )

        def wait_gather(l, a):
            pltpu.make_async_copy(
                table_hbm.at[idx_v.at[l]], rows_v.at[a], gsem.at[a]
            ).wait()

        def out_pair(l, a):
            return rowst_v.at[a], out_hbm.at[l, :, pl.ds(col0, BLK)]

        def fire_out(l, a):
            src, dst = out_pair(l, a)
            pltpu.async_copy(src, dst, osem.at[a])

        def wait_out(l, a):
            src, dst = out_pair(l, a)
            pltpu.make_async_copy(src, dst, osem.at[a]).wait()

        def transpose(a):
            src = rows_v.at[a]
            dst = rowst_v.at[a]

            def fbody(f, _):
                fs = jnp.zeros((16,), jnp.int32) + f
                for g in range(BLK // 16):
                    c = g * 16 + iota
                    v = plsc.load_gather(src, [c, fs])
                    plsc.store_scatter(dst, [fs, c], v)
                return ()

            lax.fori_loop(0, d, fbody, ())

        def step(l, a, first, last):
            if not last:
                fire_gather(l + 1, a ^ 1)
            wait_gather(l, a)
            if not first:
                wait_out(l - 2, a)
            transpose(a)
            fire_out(l, a)

        fire_gather(0, 0)
        step(0, 0, True, False)
        step(1, 1, True, False)

        def group(g, _):
            l = 2 * g
            step(l, 0, False, False)
            step(l + 1, 1, False, False)
            return ()

        lax.fori_loop(1, s // 2 - 1, group, ())

        step(s - 2, 0, False, False)
        step(s - 1, 1, False, True)
        wait_out(s - 2, 0)
        wait_out(s - 1, 1)

    return gather_kernel


def kernel(token_ids, table):
    b, s = token_ids.shape
    d = table.shape[1]
    idx_t = token_ids.T.astype(jnp.int32)
    out = _make_gather(s, b, d)(idx_t, table)
    return out.transpose(2, 0, 1)


# speedup vs baseline: 1.3100x; 1.3100x over previous
import functools

import jax
import jax.numpy as jnp
from jax import lax
from jax.experimental import pallas as pl
from jax.experimental.pallas import tpu as pltpu
from jax.experimental.pallas import tpu_sc as plsc

BLK = 128


@functools.lru_cache(maxsize=None)
def _make_gather(s: int, b: int, d: int):
    info = plsc.get_sparse_core_info()
    nc = info.num_cores
    nw = nc * info.num_subcores
    assert b == nw * BLK and d % 16 == 0
    mesh = plsc.VectorSubcoreMesh(core_axis_name="c", subcore_axis_name="s")

    @functools.partial(
        pl.kernel,
        mesh=mesh,
        out_type=jax.ShapeDtypeStruct((s, d, b), jnp.float32),
        compiler_params=pltpu.CompilerParams(
            use_tc_tiling_on_sc=False, needs_layout_passes=False
        ),
        scratch_types=[
            pltpu.VMEM((s, BLK), jnp.int32),
            pltpu.VMEM((2, BLK, d), jnp.float32),
            pltpu.VMEM((2, d, BLK), jnp.float32),
            pltpu.SemaphoreType.DMA((2,)),
            pltpu.SemaphoreType.DMA((2,)),
        ],
    )
    def gather_kernel(idx_hbm, table_hbm, out_hbm, idx_v, rows_v, rowst_v, gsem, osem):
        wid = lax.axis_index("s") * nc + lax.axis_index("c")
        col0 = wid * BLK
        pltpu.sync_copy(idx_hbm.at[:, pl.ds(col0, BLK)], idx_v)

        iota = lax.iota(jnp.int32, 16)

        def fire_gather(l, a):
            pltpu.async_copy(
                table_hbm.at[idx_v.at[l]], rows_v.at[a], gsem.at[a]
            )

        def wait_gather(l, a):
            pltpu.make_async_copy(
                table_hbm.at[idx_v.at[l]], rows_v.at[a], gsem.at[a]
            ).wait()

        def out_pair(l, a):
            return rowst_v.at[a], out_hbm.at[l, :, pl.ds(col0, BLK)]

        def fire_out(l, a):
            src, dst = out_pair(l, a)
            pltpu.async_copy(src, dst, osem.at[a])

        def wait_out(l, a):
            src, dst = out_pair(l, a)
            pltpu.make_async_copy(src, dst, osem.at[a]).wait()

        fsplats = [jnp.full((16,), f, jnp.int32) for f in range(d)]

        def transpose(a):
            src = rows_v.at[a]
            dst = rowst_v.at[a]

            def gbody(g, _):
                c = g * 16 + iota
                for f0 in range(0, d, 8):
                    vs = [
                        plsc.load_gather(src, [c, fsplats[f0 + u]])
                        for u in range(8)
                    ]
                    for u in range(8):
                        plsc.store_scatter(dst, [fsplats[f0 + u], c], vs[u])
                return ()

            lax.fori_loop(0, BLK // 16, gbody, ())

        def step(l, a, first, last):
            if not last:
                fire_gather(l + 1, a ^ 1)
            wait_gather(l, a)
            if not first:
                wait_out(l - 2, a)
            transpose(a)
            fire_out(l, a)

        fire_gather(0, 0)
        step(0, 0, True, False)
        step(1, 1, True, False)

        def group(g, _):
            l = 2 * g
            step(l, 0, False, False)
            step(l + 1, 1, False, False)
            return ()

        lax.fori_loop(1, s // 2 - 1, group, ())

        step(s - 2, 0, False, False)
        step(s - 1, 1, False, True)
        wait_out(s - 2, 0)
        wait_out(s - 1, 1)

    return gather_kernel


def kernel(token_ids, table):
    b, s = token_ids.shape
    d = table.shape[1]
    idx_t = token_ids.T.astype(jnp.int32)
    out = _make_gather(s, b, d)(idx_t, table)
    return out.transpose(2, 0, 1)


# speedup vs baseline: 1.3136x; 1.0028x over previous
import functools

import jax
import jax.numpy as jnp
from jax import lax
from jax.experimental import pallas as pl
from jax.experimental.pallas import tpu as pltpu
from jax.experimental.pallas import tpu_sc as plsc

BLK = 128


@functools.lru_cache(maxsize=None)
def _make_gather(s: int, b: int, d: int):
    info = plsc.get_sparse_core_info()
    nc = info.num_cores
    nw = nc * info.num_subcores
    assert b == nw * BLK and d % 16 == 0
    mesh = plsc.VectorSubcoreMesh(core_axis_name="c", subcore_axis_name="s")

    @functools.partial(
        pl.kernel,
        mesh=mesh,
        out_type=jax.ShapeDtypeStruct((s, d, b), jnp.float32),
        compiler_params=pltpu.CompilerParams(
            use_tc_tiling_on_sc=False, needs_layout_passes=False
        ),
        scratch_types=[
            pltpu.VMEM((s, BLK), jnp.int32),
            pltpu.VMEM((2, BLK, d), jnp.float32),
            pltpu.VMEM((2, d, BLK), jnp.float32),
            pltpu.SemaphoreType.DMA((2,)),
            pltpu.SemaphoreType.DMA((2,)),
        ],
    )
    def gather_kernel(idx_hbm, table_hbm, out_hbm, idx_v, rows_v, rowst_v, gsem, osem):
        wid = lax.axis_index("s") * nc + lax.axis_index("c")
        col0 = wid * BLK
        pltpu.sync_copy(idx_hbm.at[:, pl.ds(col0, BLK)], idx_v)

        iota = lax.iota(jnp.int32, 16)

        def fire_gather(l, a):
            pltpu.async_copy(
                table_hbm.at[idx_v.at[l]], rows_v.at[a], gsem.at[a]
            )

        def wait_gather(l, a):
            pltpu.make_async_copy(
                table_hbm.at[idx_v.at[l]], rows_v.at[a], gsem.at[a]
            ).wait()

        def out_pair(l, a):
            return rowst_v.at[a], out_hbm.at[l, :, pl.ds(col0, BLK)]

        def fire_out(l, a):
            src, dst = out_pair(l, a)
            pltpu.async_copy(src, dst, osem.at[a])

        def wait_out(l, a):
            src, dst = out_pair(l, a)
            pltpu.make_async_copy(src, dst, osem.at[a]).wait()

        fsplats = [jnp.full((16,), f, jnp.int32) for f in range(d)]

        def transpose(a):
            src = rows_v.at[a]
            dst = rowst_v.at[a]
            for g in range(BLK // 16):
                c = g * 16 + iota
                for f0 in range(0, d, 8):
                    vs = [
                        plsc.load_gather(src, [c, fsplats[f0 + u]])
                        for u in range(8)
                    ]
                    for u in range(8):
                        dst[f0 + u, pl.ds(g * 16, 16)] = vs[u]

        def step(l, a, first, last):
            if not last:
                fire_gather(l + 1, a ^ 1)
            wait_gather(l, a)
            if not first:
                wait_out(l - 2, a)
            transpose(a)
            fire_out(l, a)

        fire_gather(0, 0)
        step(0, 0, True, False)
        step(1, 1, True, False)

        def group(g, _):
            l = 2 * g
            step(l, 0, False, False)
            step(l + 1, 1, False, False)
            return ()

        lax.fori_loop(1, s // 2 - 1, group, ())

        step(s - 2, 0, False, False)
        step(s - 1, 1, False, True)
        wait_out(s - 2, 0)
        wait_out(s - 1, 1)

    return gather_kernel


def kernel(token_ids, table):
    b, s = token_ids.shape
    d = table.shape[1]
    idx_t = token_ids.T.astype(jnp.int32)
    out = _make_gather(s, b, d)(idx_t, table)
    return out.transpose(2, 0, 1)


# speedup vs baseline: 1.5884x; 1.2091x over previous
import functools

import jax
import jax.numpy as jnp
from jax import lax
from jax.experimental import pallas as pl
from jax.experimental.pallas import tpu as pltpu
from jax.experimental.pallas import tpu_sc as plsc

BLK = 128
PADD = 128


@functools.lru_cache(maxsize=None)
def _make_gather(s: int, b: int, d: int):
    info = plsc.get_sparse_core_info()
    nc = info.num_cores
    nw = nc * info.num_subcores
    assert b == nw * BLK and d % 8 == 0
    mesh = plsc.VectorSubcoreMesh(core_axis_name="c", subcore_axis_name="s")

    @functools.partial(
        pl.kernel,
        mesh=mesh,
        out_type=jax.ShapeDtypeStruct((s, d // 8, nw, 8, BLK), jnp.float32),
        compiler_params=pltpu.CompilerParams(
            use_tc_tiling_on_sc=True, needs_layout_passes=False
        ),
        scratch_types=[
            pltpu.VMEM((s, BLK), jnp.int32),
            pltpu.VMEM((2, BLK, PADD), jnp.float32),
            pltpu.VMEM((2, d // 8, 8, BLK), jnp.float32),
            pltpu.SemaphoreType.DMA((2,)),
            pltpu.SemaphoreType.DMA((2,)),
        ],
    )
    def gather_kernel(idx_hbm, table_hbm, out_hbm, idx_v, rows_v, rowst_v, gsem, osem):
        wid = lax.axis_index("s") * nc + lax.axis_index("c")
        col0 = wid * BLK
        pltpu.sync_copy(idx_hbm.at[:, pl.ds(col0, BLK)], idx_v)

        iota = lax.iota(jnp.int32, 16)
        fsplats = [jnp.full((16,), f, jnp.int32) for f in range(d)]

        def fire_gather(l, a):
            pltpu.async_copy(
                table_hbm.at[idx_v.at[l]], rows_v.at[a], gsem.at[a]
            )

        def wait_gather(l, a):
            pltpu.make_async_copy(
                table_hbm.at[idx_v.at[l]], rows_v.at[a], gsem.at[a]
            ).wait()

        def out_pair(l, a):
            return rowst_v.at[a], out_hbm.at[l, :, wid, :, :]

        def fire_out(l, a):
            src, dst = out_pair(l, a)
            pltpu.async_copy(src, dst, osem.at[a])

        def wait_out(l, a):
            src, dst = out_pair(l, a)
            pltpu.make_async_copy(src, dst, osem.at[a]).wait()

        def transpose(a):
            src = rows_v.at[a]
            dst = rowst_v.at[a]
            for g in range(BLK // 16):
                c = g * 16 + iota
                for f0 in range(0, d, 8):
                    vs = [
                        plsc.load_gather(src, [c, fsplats[f0 + u]])
                        for u in range(8)
                    ]
                    for u in range(8):
                        f = f0 + u
                        dst[f // 8, f % 8, pl.ds(g * 16, 16)] = vs[u]

        def step(l, a, first, last):
            if not last:
                fire_gather(l + 1, a ^ 1)
            wait_gather(l, a)
            if not first:
                wait_out(l - 2, a)
            transpose(a)
            fire_out(l, a)

        fire_gather(0, 0)
        step(0, 0, True, False)
        step(1, 1, True, False)

        def group(g, _):
            l = 2 * g
            step(l, 0, False, False)
            step(l + 1, 1, False, False)
            return ()

        lax.fori_loop(1, s // 2 - 1, group, ())

        step(s - 2, 0, False, False)
        step(s - 1, 1, False, True)
        wait_out(s - 2, 0)
        wait_out(s - 1, 1)

    return gather_kernel


def kernel(token_ids, table):
    b, s = token_ids.shape
    d = table.shape[1]
    nblk = b // BLK
    idx_t = token_ids.T.astype(jnp.int32)
    table_p = jnp.pad(table, ((0, 0), (0, PADD - d)))
    out5 = _make_gather(s, b, d)(idx_t, table_p)
    return out5.transpose(2, 4, 0, 1, 3).reshape(b, s, d)
